# baseline (speedup 1.0000x reference)
import functools

import jax
import jax.numpy as jnp
from jax import lax
from jax.experimental import pallas as pl
from jax.experimental.pallas import tpu as pltpu
from jax.experimental.pallas import tpu_sc as plsc

_N = 1_000_000
_CBLK = 25_600
_GRID = 16
_CT = 2 * _CBLK * _GRID

_CH0 = _CT // 128
_NCH = -(-_N // 128)
_NW = 32
_K = -(-(_NCH - _CH0) // _NW)
_KPAIR = -(-_K // 2)

_mesh = plsc.VectorSubcoreMesh(core_axis_name="c", subcore_axis_name="s")


def _tc_body(u0_ref, u1_ref, v0_ref, v1_ref, o_ref, acc_ref):
    step = pl.program_id(0)

    @pl.when(step == 0)
    def _init():
        acc_ref[...] = jnp.zeros_like(acc_ref)

    ones_row = jnp.ones((1, 8), jnp.float32)

    def block_norms(x):
        x2 = x * x
        z = x2[0:8, :] + x2[8:16, :] + x2[16:24, :] + x2[24:32, :]
        n2 = jax.lax.dot_general(
            ones_row, z, (((1,), (0,)), ((), ())),
            preferred_element_type=jnp.float32)
        return jnp.sqrt(n2)

    acc_ref[0:1, :] += (
        block_norms(u0_ref[...]) + block_norms(u1_ref[...])
        + block_norms(v0_ref[...]) + block_norms(v1_ref[...])
    )

    @pl.when(step == _GRID - 1)
    def _fin():
        o_ref[0, 0] = jnp.sum(acc_ref[0:1, :])


def _tc_part(ut, vt):
    spec0 = pl.BlockSpec((32, _CBLK), lambda i: (0, i))
    spec1 = pl.BlockSpec((32, _CBLK), lambda i: (0, _GRID + i))
    out = pl.pallas_call(
        _tc_body,
        grid=(_GRID,),
        in_specs=[spec0, spec1, spec0, spec1],
        out_specs=pl.BlockSpec(memory_space=pltpu.SMEM),
        out_shape=jax.ShapeDtypeStruct((1, 1), jnp.float32),
        scratch_shapes=[pltpu.VMEM((8, _CBLK), jnp.float32)],
    )(ut, ut, vt, vt)
    return out[0, 0]


def _sc_body(u_ref, v_ref, out_ref, buf0, buf1, accbuf, sem0, sem1):
    wid = lax.axis_index("s") * 2 + lax.axis_index("c")
    lo = _CH0 + wid * _K
    hi = jnp.minimum(lo + _K, _NCH)

    lane16 = lax.iota(jnp.int32, 16)

    def start(tab_ref, ch, buf, sem):
        c0 = jnp.minimum(ch, _NCH - 1) * 128
        pltpu.make_async_copy(tab_ref.at[:, pl.ds(c0, 128)], buf, sem).start()

    def wait(tab_ref, buf, sem):
        pltpu.make_async_copy(tab_ref.at[:, pl.ds(0, 128)], buf, sem).wait()

    def contrib(buf, ch, acc):
        c0 = ch * 128
        total = jnp.zeros((16,), jnp.float32)
        for j in range(8):
            parts = [jnp.zeros((16,), jnp.float32) for _ in range(4)]
            for r in range(32):
                x = buf[r, pl.ds(j * 16, 16)]
                parts[r % 4] = parts[r % 4] + x * x
            n2 = (parts[0] + parts[1]) + (parts[2] + parts[3])
            i = lax.bitcast_convert_type(n2, jnp.int32)
            y = lax.bitcast_convert_type(0x5F3759DF - (i >> 1), jnp.float32)
            y = y * (1.5 - 0.5 * n2 * y * y)
            y = y * (1.5 - 0.5 * n2 * y * y)
            norm = jnp.where(n2 > 0.0, n2 * y, 0.0)
            col = c0 + j * 16 + lane16
            total = total + jnp.where(col < _N, norm, 0.0)
        return acc + total * jnp.where(ch < hi, 1.0, 0.0)

    def process(tab_ref, acc0):
        start(tab_ref, lo, buf0, sem0)

        def pair_body(t, acc):
            ch0 = lo + 2 * t
            start(tab_ref, ch0 + 1, buf1, sem1)
            wait(tab_ref, buf0, sem0)
            acc = contrib(buf0, ch0, acc)
            start(tab_ref, ch0 + 2, buf0, sem0)
            wait(tab_ref, buf1, sem1)
            acc = contrib(buf1, ch0 + 1, acc)
            return acc

        acc = lax.fori_loop(0, _KPAIR, pair_body, acc0)
        wait(tab_ref, buf0, sem0)
        return acc

    acc = process(u_ref, jnp.zeros((16,), jnp.float32))
    acc = process(v_ref, acc)
    accbuf[...] = acc
    pltpu.sync_copy(accbuf, out_ref.at[wid])


_sc_part = functools.partial(
    pl.kernel,
    out_type=jax.ShapeDtypeStruct((_NW, 16), jnp.float32),
    mesh=_mesh,
    scratch_types=[
        pltpu.VMEM((32, 128), jnp.float32),
        pltpu.VMEM((32, 128), jnp.float32),
        pltpu.VMEM((16,), jnp.float32),
        pltpu.SemaphoreType.DMA,
        pltpu.SemaphoreType.DMA,
    ],
)(_sc_body)


def kernel(user_embedding, item_embedding):
    ut = user_embedding.T
    vt = item_embedding.T
    sc_partials = _sc_part(ut, vt)
    tc_total = _tc_part(ut, vt)
    return tc_total + jnp.sum(sc_partials)


# speedup vs baseline: 1.3029x; 1.3029x over previous
import jax
import jax.numpy as jnp
from jax.experimental import pallas as pl
from jax.experimental.pallas import tpu as pltpu

_N = 1_000_000
_CBLK = 25_600
_NBLK = -(-_N // _CBLK)
_SPT = 2
_GRID = _NBLK // _SPT


def _norm_sum_body(u0_ref, u1_ref, v0_ref, v1_ref, o_ref, acc_ref):
    step = pl.program_id(0)

    @pl.when(step == 0)
    def _init():
        acc_ref[...] = jnp.zeros_like(acc_ref)

    ones_row = jnp.ones((1, 8), jnp.float32)
    lane = jax.lax.broadcasted_iota(jnp.int32, (1, _CBLK), 1)

    def block_norms(x, blk_idx):
        col = lane + blk_idx * _CBLK
        x2 = x * x
        z = x2[0:8, :] + x2[8:16, :] + x2[16:24, :] + x2[24:32, :]
        n2 = jax.lax.dot_general(
            ones_row, z, (((1,), (0,)), ((), ())),
            preferred_element_type=jnp.float32)
        return jnp.where(col < _N, jnp.sqrt(n2), 0.0)

    acc_ref[0:1, :] += (
        block_norms(u0_ref[...], step)
        + block_norms(u1_ref[...], _GRID + step)
        + block_norms(v0_ref[...], step)
        + block_norms(v1_ref[...], _GRID + step)
    )

    @pl.when(step == _GRID - 1)
    def _fin():
        o_ref[0, 0] = jnp.sum(acc_ref[0:1, :])


def kernel(user_embedding, item_embedding):
    ut = user_embedding.T
    vt = item_embedding.T
    spec0 = pl.BlockSpec((32, _CBLK), lambda i: (0, i))
    spec1 = pl.BlockSpec((32, _CBLK), lambda i: (0, _GRID + i))
    out = pl.pallas_call(
        _norm_sum_body,
        grid=(_GRID,),
        in_specs=[spec0, spec1, spec0, spec1],
        out_specs=pl.BlockSpec(memory_space=pltpu.SMEM),
        out_shape=jax.ShapeDtypeStruct((1, 1), jnp.float32),
        scratch_shapes=[pltpu.VMEM((8, _CBLK), jnp.float32)],
    )(ut, ut, vt, vt)
    return out[0, 0]
